# baseline (speedup 1.0000x reference)
import dataclasses

import jax
import jax.numpy as jnp
import numpy as np
from jax import lax
from jax.experimental import pallas as pl
from jax.experimental.pallas import tpu as pltpu
from jax.experimental.pallas import tpu_sc as plsc

_B, _A, _H, _W = 4, 9, 64, 64
_P = _H * _W
_N = _A * _P
_ROWS, _LANES = 288, 128
_PRE_NMS = 6000
_K = 6016
_KROWS = 47
_POST_NMS = 300
_THRESH = 0.7
_STRIDE = 16.0
_CH = 4608
_NCHUNK = _N // _CH


def _anchor_planes_np():
    base = np.array([1.0, 1.0, 16.0, 16.0]) - 1.0
    w = base[2] - base[0] + 1.0
    h = base[3] - base[1] + 1.0
    xc, yc = base[0] + 0.5 * (w - 1.0), base[1] + 0.5 * (h - 1.0)
    sz = w * h
    r = np.array((0.5, 1.0, 2.0), dtype=np.float64)
    ws = np.round(np.sqrt(sz / r))
    hs = np.round(ws * r)

    def mk(ws_, hs_, xc_, yc_):
        ws_ = np.asarray(ws_, dtype=np.float64).reshape(-1, 1)
        hs_ = np.asarray(hs_, dtype=np.float64).reshape(-1, 1)
        return np.hstack((xc_ - 0.5 * (ws_ - 1.0), yc_ - 0.5 * (hs_ - 1.0),
                          xc_ + 0.5 * (ws_ - 1.0), yc_ + 0.5 * (hs_ - 1.0)))

    ra = mk(ws, hs, xc, yc)
    out = []
    for i in range(ra.shape[0]):
        wi = ra[i][2] - ra[i][0] + 1.0
        hi = ra[i][3] - ra[i][1] + 1.0
        xci, yci = ra[i][0] + 0.5 * (wi - 1.0), ra[i][1] + 0.5 * (hi - 1.0)
        s = np.array((8.0, 16.0, 32.0))
        out.append(mk(wi * s, hi * s, xci, yci))
    anchors9 = np.vstack(out).astype(np.float32)

    sx = np.arange(_W, dtype=np.float32) * _STRIDE
    sy = np.arange(_H, dtype=np.float32) * _STRIDE
    sxx, syy = np.meshgrid(sx, sy)
    shifts = np.stack([sxx.ravel(), syy.ravel(), sxx.ravel(), syy.ravel()],
                      axis=1)
    planes = anchors9.T[:, None, :] + shifts.T[:, :, None]
    return planes.reshape(4, _ROWS, _LANES).astype(np.float32)


def _prep_kernel(scl_ref, dx_ref, dy_ref, dw_ref, dh_ref, anc_ref, img_ref,
                 live_o, x1_o, y1_o, x2_o, y2_o):
    i32 = jnp.int32

    rr = lax.broadcasted_iota(i32, (_ROWS, _LANES), 0)
    cc = lax.broadcasted_iota(i32, (_ROWS, _LANES), 1)
    pos_i = rr * _LANES + cc

    ax1, ay1, ax2, ay2 = anc_ref[0], anc_ref[1], anc_ref[2], anc_ref[3]
    wa = ax2 - ax1 + 1.0
    ha = ay2 - ay1 + 1.0
    cxa = ax1 + 0.5 * wa
    cya = ay1 + 0.5 * ha

    dx = dx_ref[...]
    dy = dy_ref[...]
    dw = dw_ref[...]
    dh = dh_ref[...]
    pcx = dx * wa + cxa
    pcy = dy * ha + cya
    pw = jnp.exp(dw) * wa
    ph = jnp.exp(dh) * ha
    x1 = pcx - 0.5 * pw
    y1 = pcy - 0.5 * ph
    x2 = pcx + 0.5 * pw
    y2 = pcy + 0.5 * ph

    img = img_ref[...]
    im_h = img[:, 0].reshape(_B, 1, 1)
    im_w = img[:, 1].reshape(_B, 1, 1)
    x1 = jnp.clip(x1, 0.0, im_w - 1.0)
    y1 = jnp.clip(y1, 0.0, im_h - 1.0)
    x2 = jnp.clip(x2, 0.0, im_w - 1.0)
    y2 = jnp.clip(y2, 0.0, im_h - 1.0)

    scores = scl_ref[...]
    bits = lax.bitcast_convert_type(scores, i32)

    lo = jnp.zeros((_B, 1, 1), i32)
    hi = jnp.full((_B, 1, 1), 1 << 30, i32)

    def vsearch(_, lh):
        lo_, hi_ = lh
        mid = lo_ + ((hi_ - lo_) >> 1)
        cnt = jnp.sum((bits > mid).astype(i32), axis=(1, 2), keepdims=True)
        pred = cnt < _PRE_NMS
        return (jnp.where(pred, lo_, mid + 1), jnp.where(pred, mid, hi_))

    lo, hi = lax.fori_loop(0, 31, vsearch, (lo, hi))
    vstar = lo
    c_gt = jnp.sum((bits > vstar).astype(i32), axis=(1, 2), keepdims=True)
    need = _PRE_NMS - c_gt
    eqv = bits == vstar

    lo2 = jnp.zeros((_B, 1, 1), i32)
    hi2 = jnp.full((_B, 1, 1), 1 << 16, i32)

    def usearch(_, lh):
        lo_, hi_ = lh
        mid = lo_ + ((hi_ - lo_) >> 1)
        cnt = jnp.sum((eqv & (pos_i < mid)).astype(i32), axis=(1, 2),
                      keepdims=True)
        pred = cnt >= need
        return (jnp.where(pred, lo_, mid + 1), jnp.where(pred, mid, hi_))

    lo2, hi2 = lax.fori_loop(0, 17, usearch, (lo2, hi2))
    cand = (bits > vstar) | (eqv & (pos_i < lo2))

    live_o[...] = jnp.where(cand, scores, -1.0)
    x1_o[...] = x1
    y1_o[...] = y1
    x2_o[...] = x2
    y2_o[...] = y2


def _compact_kernel(live_h, x1_h, y1_h, x2_h, y2_h,
                    olv_h, ox1_h, oy1_h, ox2_h, oy2_h,
                    ia0, ia1, ia2, ia3, ia4,
                    ib0, ib1, ib2, ib3, ib4,
                    clv, cx1, cy1, cx2, cy2,
                    sema, semb):
    wid = lax.axis_index("s") * 2 + lax.axis_index("c")

    @pl.when(wid < _B)
    def _():
        b = wid
        bufs = ((ia0, ia1, ia2, ia3, ia4),
                (ib0, ib1, ib2, ib3, ib4))
        sems = (sema, semb)
        outs = (clv, cx1, cy1, cx2, cy2)
        planes = (live_h, x1_h, y1_h, x2_h, y2_h)

        def issue(ck, par):
            base = ck * _CH
            return [pltpu.async_copy(p.at[b, pl.ds(base, _CH)],
                                     bufs[par][j], sems[par])
                    for j, p in enumerate(planes)]

        def do_chunk(par, off):
            ins = bufs[par]

            def step(i, off_):
                sls = [pl.ds((i * 4 + j) * 16, 16) for j in range(4)]
                lvs = [ins[0][s] for s in sls]
                msks = [lv >= 0.0 for lv in lvs]
                cnts = [jnp.sum(m.astype(jnp.int32)) for m in msks]
                offs = [off_,
                        off_ + cnts[0],
                        off_ + cnts[0] + cnts[1],
                        off_ + cnts[0] + cnts[1] + cnts[2]]
                for j in range(4):
                    dst = pl.ds(offs[j], 16)
                    plsc.store_compressed(clv.at[dst], lvs[j], mask=msks[j])
                    plsc.store_compressed(cx1.at[dst], ins[1][sls[j]],
                                          mask=msks[j])
                    plsc.store_compressed(cy1.at[dst], ins[2][sls[j]],
                                          mask=msks[j])
                    plsc.store_compressed(cx2.at[dst], ins[3][sls[j]],
                                          mask=msks[j])
                    plsc.store_compressed(cy2.at[dst], ins[4][sls[j]],
                                          mask=msks[j])
                return offs[3] + cnts[3]

            return lax.fori_loop(0, _CH // 64, step, off)

        pend = issue(0, 0)
        off = jnp.int32(0)
        for ck in range(_NCHUNK):
            par = ck % 2
            for cp in pend:
                cp.wait()
            if ck + 1 < _NCHUNK:
                pend = issue(ck + 1, 1 - par)
            off = do_chunk(par, off)

        pad = pl.ds(_PRE_NMS, 16)
        zeros = jnp.zeros((16,), jnp.float32)
        clv[pad] = jnp.full((16,), -1.0, jnp.float32)
        cx1[pad] = zeros
        cy1[pad] = zeros
        cx2[pad] = zeros
        cy2[pad] = zeros

        ocs = [pltpu.async_copy(src, dst.at[b], sema)
               for src, dst in zip(outs, (olv_h, ox1_h, oy1_h, ox2_h,
                                          oy2_h))]
        for cp in ocs:
            cp.wait()


def _nms_kernel(lv_in, px1, py1, px2, py2,
                ox1_ref, oy1_ref, ox2_ref, oy2_ref, live_ref, area_ref):
    f32 = jnp.float32
    live_ref[...] = lv_in[...]
    area_ref[...] = ((px2[...] - px1[...] + 1.0)
                     * (py2[...] - py1[...] + 1.0))
    rr = lax.broadcasted_iota(jnp.int32, (_KROWS, _LANES), 0)
    cc = lax.broadcasted_iota(jnp.int32, (_KROWS, _LANES), 1)
    slotf = (rr * _LANES + cc).astype(f32)

    def step(t, _):
        sl = live_ref[...]
        m = jnp.max(sl, axis=(1, 2), keepdims=True)
        act = m > -0.5
        actf = act.astype(f32)
        eq = sl == m
        pstar = jnp.min(jnp.where(eq, slotf, 1e9), axis=(1, 2), keepdims=True)
        oh = (eq & (slotf == pstar)).astype(f32)

        cx1 = px1[...]
        cy1 = py1[...]
        cx2 = px2[...]
        cy2 = py2[...]
        car = area_ref[...]
        wx1 = jnp.sum(oh * cx1, axis=(1, 2), keepdims=True)
        wy1 = jnp.sum(oh * cy1, axis=(1, 2), keepdims=True)
        wx2 = jnp.sum(oh * cx2, axis=(1, 2), keepdims=True)
        wy2 = jnp.sum(oh * cy2, axis=(1, 2), keepdims=True)
        war = (wx2 - wx1 + 1.0) * (wy2 - wy1 + 1.0)

        ox1_ref[t] = (wx1 * actf).reshape(_B, 1)
        oy1_ref[t] = (wy1 * actf).reshape(_B, 1)
        ox2_ref[t] = (wx2 * actf).reshape(_B, 1)
        oy2_ref[t] = (wy2 * actf).reshape(_B, 1)

        xx1 = jnp.maximum(cx1, wx1)
        yy1 = jnp.maximum(cy1, wy1)
        xx2 = jnp.minimum(cx2, wx2)
        yy2 = jnp.minimum(cy2, wy2)
        iw = jnp.maximum(xx2 - xx1 + 1.0, 0.0)
        ih = jnp.maximum(yy2 - yy1 + 1.0, 0.0)
        inter = iw * ih
        union = car + war - inter
        sup = (inter > _THRESH * union) & act
        live_ref[...] = jnp.where(sup, -1.0, sl)
        return 0

    lax.fori_loop(0, _POST_NMS, step, 0)


@jax.jit
def kernel(x_cls, x_loc, img_info):
    f32 = jnp.float32
    nshape = (_B, _ROWS, _LANES)
    scores = (x_cls[:, _A:, :, :].reshape(_B, _A, _P)
              .transpose(0, 2, 1).reshape(nshape))
    dx = (x_loc[:, 0::4, :, :].reshape(_B, _A, _P)
          .transpose(0, 2, 1).reshape(nshape))
    dy = (x_loc[:, 1::4, :, :].reshape(_B, _A, _P)
          .transpose(0, 2, 1).reshape(nshape))
    dw = (x_loc[:, 2::4, :, :].reshape(_B, _A, _P)
          .transpose(0, 2, 1).reshape(nshape))
    dh = (x_loc[:, 3::4, :, :].reshape(_B, _A, _P)
          .transpose(0, 2, 1).reshape(nshape))
    anc = jnp.asarray(_anchor_planes_np())

    plane = jax.ShapeDtypeStruct(nshape, f32)
    live, x1, y1, x2, y2 = pl.pallas_call(
        _prep_kernel,
        out_shape=[plane] * 5,
    )(scores, dx, dy, dw, dh, anc, img_info)

    flat = lambda t: t.reshape(_B, _N)
    cplane = jax.ShapeDtypeStruct((_B, _K), f32)
    cp = pltpu.CompilerParams()
    if "needs_layout_passes" in pltpu.CompilerParams.__dataclass_fields__:
        cp = dataclasses.replace(cp, needs_layout_passes=False)
    compact = pl.kernel(
        _compact_kernel,
        out_type=[cplane] * 5,
        mesh=plsc.VectorSubcoreMesh(core_axis_name="c", subcore_axis_name="s",
                                    num_cores=2, num_subcores=16),
        scratch_types=[pltpu.VMEM((_CH,), f32)] * 10
                      + [pltpu.VMEM((_K,), f32)] * 5
                      + [pltpu.SemaphoreType.DMA] * 2,
        compiler_params=cp,
    )
    clive, ccx1, ccy1, ccx2, ccy2 = compact(
        flat(live), flat(x1), flat(y1), flat(x2), flat(y2))

    shp = lambda t: t.reshape(_B, _KROWS, _LANES)
    out_sds = [jax.ShapeDtypeStruct((_POST_NMS, _B, 1), f32)] * 4
    ox1, oy1, ox2, oy2 = pl.pallas_call(
        _nms_kernel,
        out_shape=out_sds,
        scratch_shapes=[pltpu.VMEM((_B, _KROWS, _LANES), f32)] * 2,
    )(shp(clive), shp(ccx1), shp(ccy1), shp(ccx2), shp(ccy2))

    sel = jnp.concatenate([ox1, oy1, ox2, oy2], axis=2)
    sel = jnp.transpose(sel, (1, 0, 2))
    col0 = jnp.broadcast_to(
        jnp.arange(_B, dtype=f32)[:, None, None], (_B, _POST_NMS, 1))
    return jnp.concatenate([col0, sel], axis=2)


# speedup vs baseline: 1.0211x; 1.0211x over previous
import dataclasses

import jax
import jax.numpy as jnp
import numpy as np
from jax import lax
from jax.experimental import pallas as pl
from jax.experimental.pallas import tpu as pltpu
from jax.experimental.pallas import tpu_sc as plsc

_B, _A, _H, _W = 4, 9, 64, 64
_P = _H * _W
_N = _A * _P
_ROWS, _LANES = 288, 128
_PRE_NMS = 6000
_K = 6016
_KROWS = 47
_POST_NMS = 300
_THRESH = 0.7
_STRIDE = 16.0
_CH = 4608
_NCHUNK = _N // _CH


def _anchor_planes_np():
    base = np.array([1.0, 1.0, 16.0, 16.0]) - 1.0
    w = base[2] - base[0] + 1.0
    h = base[3] - base[1] + 1.0
    xc, yc = base[0] + 0.5 * (w - 1.0), base[1] + 0.5 * (h - 1.0)
    sz = w * h
    r = np.array((0.5, 1.0, 2.0), dtype=np.float64)
    ws = np.round(np.sqrt(sz / r))
    hs = np.round(ws * r)

    def mk(ws_, hs_, xc_, yc_):
        ws_ = np.asarray(ws_, dtype=np.float64).reshape(-1, 1)
        hs_ = np.asarray(hs_, dtype=np.float64).reshape(-1, 1)
        return np.hstack((xc_ - 0.5 * (ws_ - 1.0), yc_ - 0.5 * (hs_ - 1.0),
                          xc_ + 0.5 * (ws_ - 1.0), yc_ + 0.5 * (hs_ - 1.0)))

    ra = mk(ws, hs, xc, yc)
    out = []
    for i in range(ra.shape[0]):
        wi = ra[i][2] - ra[i][0] + 1.0
        hi = ra[i][3] - ra[i][1] + 1.0
        xci, yci = ra[i][0] + 0.5 * (wi - 1.0), ra[i][1] + 0.5 * (hi - 1.0)
        s = np.array((8.0, 16.0, 32.0))
        out.append(mk(wi * s, hi * s, xci, yci))
    anchors9 = np.vstack(out).astype(np.float32)

    sx = np.arange(_W, dtype=np.float32) * _STRIDE
    sy = np.arange(_H, dtype=np.float32) * _STRIDE
    sxx, syy = np.meshgrid(sx, sy)
    shifts = np.stack([sxx.ravel(), syy.ravel(), sxx.ravel(), syy.ravel()],
                      axis=1)
    planes = anchors9.T[:, :, None] + shifts.T[:, None, :]
    return planes.reshape(4, _ROWS, _LANES).astype(np.float32)


def _prep_kernel(scl_ref, dx_ref, dy_ref, dw_ref, dh_ref, anc_ref, img_ref,
                 live_o, x1_o, y1_o, x2_o, y2_o):
    i32 = jnp.int32

    rr = lax.broadcasted_iota(i32, (_ROWS, _LANES), 0)
    cc = lax.broadcasted_iota(i32, (_ROWS, _LANES), 1)
    flat = rr * _LANES + cc
    pos_i = (flat & (_P - 1)) * _A + (flat >> 12)

    ax1, ay1, ax2, ay2 = anc_ref[0], anc_ref[1], anc_ref[2], anc_ref[3]
    wa = ax2 - ax1 + 1.0
    ha = ay2 - ay1 + 1.0
    cxa = ax1 + 0.5 * wa
    cya = ay1 + 0.5 * ha

    dx = dx_ref[...]
    dy = dy_ref[...]
    dw = dw_ref[...]
    dh = dh_ref[...]
    pcx = dx * wa + cxa
    pcy = dy * ha + cya
    pw = jnp.exp(dw) * wa
    ph = jnp.exp(dh) * ha
    x1 = pcx - 0.5 * pw
    y1 = pcy - 0.5 * ph
    x2 = pcx + 0.5 * pw
    y2 = pcy + 0.5 * ph

    img = img_ref[...]
    im_h = img[:, 0].reshape(_B, 1, 1)
    im_w = img[:, 1].reshape(_B, 1, 1)
    x1 = jnp.clip(x1, 0.0, im_w - 1.0)
    y1 = jnp.clip(y1, 0.0, im_h - 1.0)
    x2 = jnp.clip(x2, 0.0, im_w - 1.0)
    y2 = jnp.clip(y2, 0.0, im_h - 1.0)

    scores = scl_ref[...]
    bits = lax.bitcast_convert_type(scores, i32)

    lo = jnp.zeros((_B, 1, 1), i32)
    hi = jnp.full((_B, 1, 1), 1 << 30, i32)

    def vsearch(_, lh):
        lo_, hi_ = lh
        mid = lo_ + ((hi_ - lo_) >> 1)
        cnt = jnp.sum((bits > mid).astype(i32), axis=(1, 2), keepdims=True)
        pred = cnt < _PRE_NMS
        return (jnp.where(pred, lo_, mid + 1), jnp.where(pred, mid, hi_))

    lo, hi = lax.fori_loop(0, 31, vsearch, (lo, hi))
    vstar = lo
    c_gt = jnp.sum((bits > vstar).astype(i32), axis=(1, 2), keepdims=True)
    need = _PRE_NMS - c_gt
    eqv = bits == vstar

    lo2 = jnp.zeros((_B, 1, 1), i32)
    hi2 = jnp.full((_B, 1, 1), 1 << 16, i32)

    def usearch(_, lh):
        lo_, hi_ = lh
        mid = lo_ + ((hi_ - lo_) >> 1)
        cnt = jnp.sum((eqv & (pos_i < mid)).astype(i32), axis=(1, 2),
                      keepdims=True)
        pred = cnt >= need
        return (jnp.where(pred, lo_, mid + 1), jnp.where(pred, mid, hi_))

    lo2, hi2 = lax.fori_loop(0, 17, usearch, (lo2, hi2))
    cand = (bits > vstar) | (eqv & (pos_i < lo2))

    live_o[...] = jnp.where(cand, scores, -1.0)
    x1_o[...] = x1
    y1_o[...] = y1
    x2_o[...] = x2
    y2_o[...] = y2


def _compact_kernel(live_h, x1_h, y1_h, x2_h, y2_h, ga_h, gp_h,
                    olv_h, ox1_h, oy1_h, ox2_h, oy2_h,
                    ia0, ia1, ia2, ia3, ia4,
                    ib0, ib1, ib2, ib3, ib4,
                    clv, cx1, cy1, cx2, cy2,
                    ga_v, gp_v, sema, semb):
    wid = lax.axis_index("s") * 2 + lax.axis_index("c")

    @pl.when(wid < _B)
    def _():
        b = wid
        bufs = ((ia0, ia1, ia2, ia3, ia4),
                (ib0, ib1, ib2, ib3, ib4))
        sems = (sema, semb)
        outs = (clv, cx1, cy1, cx2, cy2)
        planes = (live_h, x1_h, y1_h, x2_h, y2_h)

        pltpu.sync_copy(ga_h, ga_v)
        pltpu.sync_copy(gp_h, gp_v)

        def issue(ck, par):
            base = ck * (_CH // _A)
            return [pltpu.async_copy(p.at[b, :, pl.ds(base, _CH // _A)],
                                     bufs[par][j], sems[par])
                    for j, p in enumerate(planes)]

        def do_chunk(par, off):
            ins = bufs[par]

            def step(i, off_):
                sls = [pl.ds((i * 4 + j) * 16, 16) for j in range(4)]
                gas = [ga_v[s] for s in sls]
                gps = [gp_v[s] for s in sls]
                lvs = [plsc.load_gather(ins[0], [a_, p_])
                       for a_, p_ in zip(gas, gps)]
                msks = [lv >= 0.0 for lv in lvs]
                cnts = [jnp.sum(m.astype(jnp.int32)) for m in msks]
                offs = [off_,
                        off_ + cnts[0],
                        off_ + cnts[0] + cnts[1],
                        off_ + cnts[0] + cnts[1] + cnts[2]]
                for j in range(4):
                    dst = pl.ds(offs[j], 16)
                    plsc.store_compressed(clv.at[dst], lvs[j], mask=msks[j])
                    for plane_v, out_v in ((ins[1], cx1), (ins[2], cy1),
                                           (ins[3], cx2), (ins[4], cy2)):
                        plsc.store_compressed(
                            out_v.at[dst],
                            plsc.load_gather(plane_v, [gas[j], gps[j]]),
                            mask=msks[j])
                return offs[3] + cnts[3]

            return lax.fori_loop(0, _CH // 64, step, off)

        pend = issue(0, 0)
        off = jnp.int32(0)
        for ck in range(_NCHUNK):
            par = ck % 2
            for cp in pend:
                cp.wait()
            if ck + 1 < _NCHUNK:
                pend = issue(ck + 1, 1 - par)
            off = do_chunk(par, off)

        pad = pl.ds(_PRE_NMS, 16)
        zeros = jnp.zeros((16,), jnp.float32)
        clv[pad] = jnp.full((16,), -1.0, jnp.float32)
        cx1[pad] = zeros
        cy1[pad] = zeros
        cx2[pad] = zeros
        cy2[pad] = zeros

        ocs = [pltpu.async_copy(src, dst.at[b], sema)
               for src, dst in zip(outs, (olv_h, ox1_h, oy1_h, ox2_h,
                                          oy2_h))]
        for cp in ocs:
            cp.wait()


def _nms_kernel(lv_in, px1, py1, px2, py2,
                ox1_ref, oy1_ref, ox2_ref, oy2_ref, live_ref, area_ref):
    f32 = jnp.float32
    live_ref[...] = lv_in[...]
    area_ref[...] = ((px2[...] - px1[...] + 1.0)
                     * (py2[...] - py1[...] + 1.0))
    rr = lax.broadcasted_iota(jnp.int32, (_KROWS, _LANES), 0)
    cc = lax.broadcasted_iota(jnp.int32, (_KROWS, _LANES), 1)
    slotf = (rr * _LANES + cc).astype(f32)

    def step(t, _):
        sl = live_ref[...]
        m = jnp.max(sl, axis=(1, 2), keepdims=True)
        act = m > -0.5
        actf = act.astype(f32)
        eq = sl == m
        pstar = jnp.min(jnp.where(eq, slotf, 1e9), axis=(1, 2), keepdims=True)
        oh = (eq & (slotf == pstar)).astype(f32)

        cx1 = px1[...]
        cy1 = py1[...]
        cx2 = px2[...]
        cy2 = py2[...]
        car = area_ref[...]
        wx1 = jnp.sum(oh * cx1, axis=(1, 2), keepdims=True)
        wy1 = jnp.sum(oh * cy1, axis=(1, 2), keepdims=True)
        wx2 = jnp.sum(oh * cx2, axis=(1, 2), keepdims=True)
        wy2 = jnp.sum(oh * cy2, axis=(1, 2), keepdims=True)
        war = (wx2 - wx1 + 1.0) * (wy2 - wy1 + 1.0)

        ox1_ref[t] = (wx1 * actf).reshape(_B, 1)
        oy1_ref[t] = (wy1 * actf).reshape(_B, 1)
        ox2_ref[t] = (wx2 * actf).reshape(_B, 1)
        oy2_ref[t] = (wy2 * actf).reshape(_B, 1)

        xx1 = jnp.maximum(cx1, wx1)
        yy1 = jnp.maximum(cy1, wy1)
        xx2 = jnp.minimum(cx2, wx2)
        yy2 = jnp.minimum(cy2, wy2)
        iw = jnp.maximum(xx2 - xx1 + 1.0, 0.0)
        ih = jnp.maximum(yy2 - yy1 + 1.0, 0.0)
        inter = iw * ih
        union = car + war - inter
        sup = (inter > _THRESH * union) & act
        live_ref[...] = jnp.where(sup, -1.0, sl)
        return 0

    lax.fori_loop(0, _POST_NMS, step, 0)


@jax.jit
def kernel(x_cls, x_loc, img_info):
    f32 = jnp.float32
    nshape = (_B, _ROWS, _LANES)
    scores = x_cls[:, _A:, :, :].reshape(nshape)
    dx = x_loc[:, 0::4, :, :].reshape(nshape)
    dy = x_loc[:, 1::4, :, :].reshape(nshape)
    dw = x_loc[:, 2::4, :, :].reshape(nshape)
    dh = x_loc[:, 3::4, :, :].reshape(nshape)
    anc = jnp.asarray(_anchor_planes_np())
    li = np.arange(_CH)
    ga_idx = jnp.asarray((li % _A).astype(np.int32))
    gp_idx = jnp.asarray((li // _A).astype(np.int32))

    plane = jax.ShapeDtypeStruct(nshape, f32)
    live, x1, y1, x2, y2 = pl.pallas_call(
        _prep_kernel,
        out_shape=[plane] * 5,
    )(scores, dx, dy, dw, dh, anc, img_info)

    flat = lambda t: t.reshape(_B, _A, _P)
    cplane = jax.ShapeDtypeStruct((_B, _K), f32)
    cp = pltpu.CompilerParams()
    if "needs_layout_passes" in pltpu.CompilerParams.__dataclass_fields__:
        cp = dataclasses.replace(cp, needs_layout_passes=False)
    compact = pl.kernel(
        _compact_kernel,
        out_type=[cplane] * 5,
        mesh=plsc.VectorSubcoreMesh(core_axis_name="c", subcore_axis_name="s",
                                    num_cores=2, num_subcores=16),
        scratch_types=[pltpu.VMEM((_A, _CH // _A), f32)] * 10
                      + [pltpu.VMEM((_K,), f32)] * 5
                      + [pltpu.VMEM((_CH,), jnp.int32)] * 2
                      + [pltpu.SemaphoreType.DMA] * 2,
        compiler_params=cp,
    )
    clive, ccx1, ccy1, ccx2, ccy2 = compact(
        flat(live), flat(x1), flat(y1), flat(x2), flat(y2), ga_idx, gp_idx)

    shp = lambda t: t.reshape(_B, _KROWS, _LANES)
    out_sds = [jax.ShapeDtypeStruct((_POST_NMS, _B, 1), f32)] * 4
    ox1, oy1, ox2, oy2 = pl.pallas_call(
        _nms_kernel,
        out_shape=out_sds,
        scratch_shapes=[pltpu.VMEM((_B, _KROWS, _LANES), f32)] * 2,
    )(shp(clive), shp(ccx1), shp(ccy1), shp(ccx2), shp(ccy2))

    sel = jnp.concatenate([ox1, oy1, ox2, oy2], axis=2)
    sel = jnp.transpose(sel, (1, 0, 2))
    col0 = jnp.broadcast_to(
        jnp.arange(_B, dtype=f32)[:, None, None], (_B, _POST_NMS, 1))
    return jnp.concatenate([col0, sel], axis=2)


# speedup vs baseline: 1.1995x; 1.1747x over previous
import dataclasses
import functools

import jax
import jax.numpy as jnp
import numpy as np
from jax import lax
from jax.experimental import pallas as pl
from jax.experimental.pallas import tpu as pltpu
from jax.experimental.pallas import tpu_sc as plsc

_B, _A, _H, _W = 4, 9, 64, 64
_P = _H * _W
_N = _A * _P
_ROWS, _LANES = 288, 128
_PRE_NMS = 6000
_K = 6016
_KROWS = 47
_POST_NMS = 300
_THRESH = 0.7
_STRIDE = 16.0
_CH = 4608
_NCHUNK = _N // _CH


def _anchor_planes_np():
    base = np.array([1.0, 1.0, 16.0, 16.0]) - 1.0
    w = base[2] - base[0] + 1.0
    h = base[3] - base[1] + 1.0
    xc, yc = base[0] + 0.5 * (w - 1.0), base[1] + 0.5 * (h - 1.0)
    sz = w * h
    r = np.array((0.5, 1.0, 2.0), dtype=np.float64)
    ws = np.round(np.sqrt(sz / r))
    hs = np.round(ws * r)

    def mk(ws_, hs_, xc_, yc_):
        ws_ = np.asarray(ws_, dtype=np.float64).reshape(-1, 1)
        hs_ = np.asarray(hs_, dtype=np.float64).reshape(-1, 1)
        return np.hstack((xc_ - 0.5 * (ws_ - 1.0), yc_ - 0.5 * (hs_ - 1.0),
                          xc_ + 0.5 * (ws_ - 1.0), yc_ + 0.5 * (hs_ - 1.0)))

    ra = mk(ws, hs, xc, yc)
    out = []
    for i in range(ra.shape[0]):
        wi = ra[i][2] - ra[i][0] + 1.0
        hi = ra[i][3] - ra[i][1] + 1.0
        xci, yci = ra[i][0] + 0.5 * (wi - 1.0), ra[i][1] + 0.5 * (hi - 1.0)
        s = np.array((8.0, 16.0, 32.0))
        out.append(mk(wi * s, hi * s, xci, yci))
    anchors9 = np.vstack(out).astype(np.float32)

    sx = np.arange(_W, dtype=np.float32) * _STRIDE
    sy = np.arange(_H, dtype=np.float32) * _STRIDE
    sxx, syy = np.meshgrid(sx, sy)
    shifts = np.stack([sxx.ravel(), syy.ravel(), sxx.ravel(), syy.ravel()],
                      axis=1)
    planes = anchors9.T[:, :, None] + shifts.T[:, None, :]
    return planes.reshape(4, _ROWS, _LANES).astype(np.float32)


def _pos_np():
    f = np.arange(_N)
    a = f >> 12
    p = f & (_P - 1)
    return (p * _A + a).astype(np.float32)


def _prep_kernel(scl_ref, dx_ref, dy_ref, dw_ref, dh_ref, anc_ref, img_ref,
                 live_o, x1_o, y1_o, x2_o, y2_o):
    i32 = jnp.int32

    rr = lax.broadcasted_iota(i32, (_ROWS, _LANES), 0)
    cc = lax.broadcasted_iota(i32, (_ROWS, _LANES), 1)
    flat = rr * _LANES + cc
    pos_i = (flat & (_P - 1)) * _A + (flat >> 12)

    ax1, ay1, ax2, ay2 = anc_ref[0], anc_ref[1], anc_ref[2], anc_ref[3]
    wa = ax2 - ax1 + 1.0
    ha = ay2 - ay1 + 1.0
    cxa = ax1 + 0.5 * wa
    cya = ay1 + 0.5 * ha

    dx = dx_ref[...]
    dy = dy_ref[...]
    dw = dw_ref[...]
    dh = dh_ref[...]
    pcx = dx * wa + cxa
    pcy = dy * ha + cya
    pw = jnp.exp(dw) * wa
    ph = jnp.exp(dh) * ha
    x1 = pcx - 0.5 * pw
    y1 = pcy - 0.5 * ph
    x2 = pcx + 0.5 * pw
    y2 = pcy + 0.5 * ph

    img = img_ref[...]
    im_h = img[:, 0].reshape(_B, 1, 1)
    im_w = img[:, 1].reshape(_B, 1, 1)
    x1 = jnp.clip(x1, 0.0, im_w - 1.0)
    y1 = jnp.clip(y1, 0.0, im_h - 1.0)
    x2 = jnp.clip(x2, 0.0, im_w - 1.0)
    y2 = jnp.clip(y2, 0.0, im_h - 1.0)

    scores = scl_ref[...]
    bits = lax.bitcast_convert_type(scores, i32)

    lo = jnp.zeros((_B, 1, 1), i32)
    hi = jnp.full((_B, 1, 1), 1 << 30, i32)

    def vsearch(_, lh):
        lo_, hi_ = lh
        mid = lo_ + ((hi_ - lo_) >> 1)
        cnt = jnp.sum((bits > mid).astype(i32), axis=(1, 2), keepdims=True)
        pred = cnt < _PRE_NMS
        return (jnp.where(pred, lo_, mid + 1), jnp.where(pred, mid, hi_))

    lo, hi = lax.fori_loop(0, 31, vsearch, (lo, hi))
    vstar = lo
    c_gt = jnp.sum((bits > vstar).astype(i32), axis=(1, 2), keepdims=True)
    need = _PRE_NMS - c_gt
    eqv = bits == vstar

    lo2 = jnp.zeros((_B, 1, 1), i32)
    hi2 = jnp.full((_B, 1, 1), 1 << 16, i32)

    def usearch(_, lh):
        lo_, hi_ = lh
        mid = lo_ + ((hi_ - lo_) >> 1)
        cnt = jnp.sum((eqv & (pos_i < mid)).astype(i32), axis=(1, 2),
                      keepdims=True)
        pred = cnt >= need
        return (jnp.where(pred, lo_, mid + 1), jnp.where(pred, mid, hi_))

    lo2, hi2 = lax.fori_loop(0, 17, usearch, (lo2, hi2))
    cand = (bits > vstar) | (eqv & (pos_i < lo2))

    live_o[...] = jnp.where(cand, scores, -1.0)
    x1_o[...] = x1
    y1_o[...] = y1
    x2_o[...] = x2
    y2_o[...] = y2


def _compact_kernel(live_h, x1_h, y1_h, x2_h, y2_h, pos_h,
                    olv_h, ox1_h, oy1_h, ox2_h, oy2_h, opo_h,
                    ia0, ia1, ia2, ia3, ia4, ia5,
                    ib0, ib1, ib2, ib3, ib4, ib5,
                    clv, cx1, cy1, cx2, cy2, cpo,
                    sema, semb):
    wid = lax.axis_index("s") * 2 + lax.axis_index("c")

    @pl.when(wid < _B)
    def _():
        b = wid
        bufs = ((ia0, ia1, ia2, ia3, ia4, ia5),
                (ib0, ib1, ib2, ib3, ib4, ib5))
        sems = (sema, semb)
        outs = (clv, cx1, cy1, cx2, cy2, cpo)
        planes = (live_h, x1_h, y1_h, x2_h, y2_h)

        def issue(ck, par):
            base = ck * _CH
            cps = [pltpu.async_copy(p.at[b, pl.ds(base, _CH)],
                                    bufs[par][j], sems[par])
                   for j, p in enumerate(planes)]
            cps.append(pltpu.async_copy(pos_h.at[pl.ds(base, _CH)],
                                        bufs[par][5], sems[par]))
            return cps

        def do_chunk(par, off):
            ins = bufs[par]

            def step(i, off_):
                sls = [pl.ds((i * 4 + j) * 16, 16) for j in range(4)]
                lvs = [ins[0][s] for s in sls]
                msks = [lv >= 0.0 for lv in lvs]
                cnts = [jnp.sum(m.astype(jnp.int32)) for m in msks]
                offs = [off_,
                        off_ + cnts[0],
                        off_ + cnts[0] + cnts[1],
                        off_ + cnts[0] + cnts[1] + cnts[2]]
                for j in range(4):
                    dst = pl.ds(offs[j], 16)
                    plsc.store_compressed(clv.at[dst], lvs[j], mask=msks[j])
                    for src_v, out_v in ((ins[1], cx1), (ins[2], cy1),
                                         (ins[3], cx2), (ins[4], cy2),
                                         (ins[5], cpo)):
                        plsc.store_compressed(out_v.at[dst], src_v[sls[j]],
                                              mask=msks[j])
                return offs[3] + cnts[3]

            return lax.fori_loop(0, _CH // 64, step, off)

        pend = issue(0, 0)
        off = jnp.int32(0)
        for ck in range(_NCHUNK):
            par = ck % 2
            for cp in pend:
                cp.wait()
            if ck + 1 < _NCHUNK:
                pend = issue(ck + 1, 1 - par)
            off = do_chunk(par, off)

        pad = pl.ds(_PRE_NMS, 16)
        zeros = jnp.zeros((16,), jnp.float32)
        clv[pad] = jnp.full((16,), -1.0, jnp.float32)
        cx1[pad] = zeros
        cy1[pad] = zeros
        cx2[pad] = zeros
        cy2[pad] = zeros
        cpo[pad] = zeros

        ocs = [pltpu.async_copy(src, dst.at[b], sema)
               for src, dst in zip(outs, (olv_h, ox1_h, oy1_h, ox2_h,
                                          oy2_h, opo_h))]
        for cp in ocs:
            cp.wait()


def _nms_kernel(lv_in, px1, py1, px2, py2, ppo,
                ox1_ref, oy1_ref, ox2_ref, oy2_ref, live_ref, area_ref):
    f32 = jnp.float32
    live_ref[...] = lv_in[...]
    area_ref[...] = ((px2[...] - px1[...] + 1.0)
                     * (py2[...] - py1[...] + 1.0))

    def step(t, _):
        sl = live_ref[...]
        m = jnp.max(sl, axis=(1, 2), keepdims=True)
        act = m > -0.5
        actf = act.astype(f32)
        eq = sl == m
        posf = ppo[...]
        pstar = jnp.min(jnp.where(eq, posf, 1e9), axis=(1, 2), keepdims=True)
        oh = (eq & (posf == pstar)).astype(f32)

        cx1 = px1[...]
        cy1 = py1[...]
        cx2 = px2[...]
        cy2 = py2[...]
        car = area_ref[...]
        wx1 = jnp.sum(oh * cx1, axis=(1, 2), keepdims=True)
        wy1 = jnp.sum(oh * cy1, axis=(1, 2), keepdims=True)
        wx2 = jnp.sum(oh * cx2, axis=(1, 2), keepdims=True)
        wy2 = jnp.sum(oh * cy2, axis=(1, 2), keepdims=True)
        war = jnp.sum(oh * car, axis=(1, 2), keepdims=True)

        ox1_ref[t] = (wx1 * actf).reshape(_B, 1)
        oy1_ref[t] = (wy1 * actf).reshape(_B, 1)
        ox2_ref[t] = (wx2 * actf).reshape(_B, 1)
        oy2_ref[t] = (wy2 * actf).reshape(_B, 1)

        xx1 = jnp.maximum(cx1, wx1)
        yy1 = jnp.maximum(cy1, wy1)
        xx2 = jnp.minimum(cx2, wx2)
        yy2 = jnp.minimum(cy2, wy2)
        iw = jnp.maximum(xx2 - xx1 + 1.0, 0.0)
        ih = jnp.maximum(yy2 - yy1 + 1.0, 0.0)
        inter = iw * ih
        union = car + war - inter
        sup = (inter > _THRESH * union) & act
        live_ref[...] = jnp.where(sup, -1.0, sl)
        return 0

    lax.fori_loop(0, _POST_NMS, step, 0)


@jax.jit
def kernel(x_cls, x_loc, img_info):
    f32 = jnp.float32
    scores = x_cls[:, _A:, :, :].reshape(_B, _ROWS, _LANES)
    dx = x_loc[:, 0::4, :, :].reshape(_B, _ROWS, _LANES)
    dy = x_loc[:, 1::4, :, :].reshape(_B, _ROWS, _LANES)
    dw = x_loc[:, 2::4, :, :].reshape(_B, _ROWS, _LANES)
    dh = x_loc[:, 3::4, :, :].reshape(_B, _ROWS, _LANES)
    anc = jnp.asarray(_anchor_planes_np())
    posf = jnp.asarray(_pos_np())

    plane = jax.ShapeDtypeStruct((_B, _ROWS, _LANES), f32)
    live, x1, y1, x2, y2 = pl.pallas_call(
        _prep_kernel,
        out_shape=[plane] * 5,
    )(scores, dx, dy, dw, dh, anc, img_info)

    flat = lambda t: t.reshape(_B, _N)
    cplane = jax.ShapeDtypeStruct((_B, _K), f32)
    cp = pltpu.CompilerParams()
    if "needs_layout_passes" in pltpu.CompilerParams.__dataclass_fields__:
        cp = dataclasses.replace(cp, needs_layout_passes=False)
    compact = pl.kernel(
        _compact_kernel,
        out_type=[cplane] * 6,
        mesh=plsc.VectorSubcoreMesh(core_axis_name="c", subcore_axis_name="s",
                                    num_cores=2, num_subcores=16),
        scratch_types=[pltpu.VMEM((_CH,), f32)] * 12
                      + [pltpu.VMEM((_K,), f32)] * 6
                      + [pltpu.SemaphoreType.DMA] * 2,
        compiler_params=cp,
    )
    clive, ccx1, ccy1, ccx2, ccy2, cpos = compact(
        flat(live), flat(x1), flat(y1), flat(x2), flat(y2), posf)

    shp = lambda t: t.reshape(_B, _KROWS, _LANES)
    out_sds = [jax.ShapeDtypeStruct((_POST_NMS, _B, 1), f32)] * 4
    ox1, oy1, ox2, oy2 = pl.pallas_call(
        _nms_kernel,
        out_shape=out_sds,
        scratch_shapes=[pltpu.VMEM((_B, _KROWS, _LANES), f32)] * 2,
    )(shp(clive), shp(ccx1), shp(ccy1), shp(ccx2), shp(ccy2), shp(cpos))

    sel = jnp.concatenate([ox1, oy1, ox2, oy2], axis=2)
    sel = jnp.transpose(sel, (1, 0, 2))
    col0 = jnp.broadcast_to(
        jnp.arange(_B, dtype=f32)[:, None, None], (_B, _POST_NMS, 1))
    return jnp.concatenate([col0, sel], axis=2)
